# baseline (speedup 1.0000x reference)
import functools

import jax
import jax.numpy as jnp
from jax import lax
from jax.experimental import pallas as pl
from jax.experimental.pallas import tpu as pltpu
from jax.experimental.pallas import tpu_sc as plsc

N_ITEMS = 16384
D = 64
NF = 5
EPS = 1e-5

NC = 2
NS = 16
NW = NC * NS
BPW = N_ITEMS // NW
CHUNK = 128
NCH = BPW // CHUNK
U = 4

_MAGIC = 0x5F3759DF


def _row_norm(h, mvec, y, gvs, bevs):
    out = []
    for j in range(4):
        out.append((h[j] - mvec) * y * gvs[j] + bevs[j])
    return out


def _encoder_body(sid_h, bid_h, cont_h, st_h, bt_h, wt_h, b_h, g_h, be_h, out_h,
                  sid_v, bid_v, cont_v, srow_v, brow_v, wt_v, b_v, g_v, be_v, out_v,
                  sem_s, sem_b):
    wid = lax.axis_index("s") * NC + lax.axis_index("c")
    base = wid * BPW
    rbase = wid * NCH

    pltpu.sync_copy(sid_h.at[pl.ds(rbase, NCH)], sid_v)
    pltpu.sync_copy(bid_h.at[pl.ds(rbase, NCH)], bid_v)
    pltpu.sync_copy(cont_h.at[pl.ds(base * 8, BPW * 8)], cont_v.at[pl.ds(0, BPW * 8)])
    pltpu.sync_copy(wt_h, wt_v)
    pltpu.sync_copy(b_h, b_v)
    pltpu.sync_copy(g_h, g_v)
    pltpu.sync_copy(be_h, be_v)

    copies = []
    for j in range(NCH):
        copies.append(pltpu.async_copy(
            st_h.at[sid_v.at[j]], srow_v.at[pl.ds(j * CHUNK, CHUNK)], sem_s))
        copies.append(pltpu.async_copy(
            bt_h.at[bid_v.at[j]], brow_v.at[pl.ds(j * CHUNK, CHUNK)], sem_b))
    for cp in copies:
        cp.wait()

    wrows = [[wt_v[k, pl.ds(16 * j, 16)] for j in range(4)] for k in range(NF)]
    bvs = [b_v[pl.ds(16 * j, 16)] for j in range(4)]
    gvs = [g_v[pl.ds(16 * j, 16)] for j in range(4)]
    bevs = [be_v[pl.ds(16 * j, 16)] for j in range(4)]
    third = jnp.float32(1.0 / 3.0)
    inv_d = jnp.float32(1.0 / D)

    def body(it, carry):
        for u in range(U):
            i = it * U + u
            cv = cont_v[pl.ds(pl.multiple_of(i * 8, 8), 16)]
            cf = [cv[k] for k in range(NF)]
            h = []
            for j in range(4):
                hv = srow_v[i, pl.ds(16 * j, 16)] + brow_v[i, pl.ds(16 * j, 16)] + bvs[j]
                for k in range(NF):
                    hv = hv + cf[k] * wrows[k][j]
                h.append(hv * third)
            tot = jnp.sum(h[0] + h[1] + h[2] + h[3])
            sq = jnp.sum(h[0] * h[0] + h[1] * h[1] + h[2] * h[2] + h[3] * h[3])
            mean = tot * inv_d
            var = sq * inv_d - mean * mean + jnp.float32(EPS)
            mvec = jnp.full((16,), mean, jnp.float32)
            vvec = jnp.full((16,), var, jnp.float32)
            iv = plsc.bitcast(vvec, jnp.int32)
            iv = _MAGIC - lax.shift_right_arithmetic(iv, 1)
            y = plsc.bitcast(iv, jnp.float32)
            for _ in range(3):
                y = y * (jnp.float32(1.5) - jnp.float32(0.5) * vvec * y * y)
            outs = _row_norm(h, mvec, y, gvs, bevs)
            for j in range(4):
                out_v[i, pl.ds(16 * j, 16)] = outs[j]
        return carry

    lax.fori_loop(0, BPW // U, body, 0)
    pltpu.sync_copy(out_v, out_h.at[pl.ds(base, BPW)])


def kernel(style_ids, brewer_ids, cont_feats, style_table, brewer_table, W, b, gamma, beta):
    sid2 = style_ids.reshape(N_ITEMS // 128, 128)
    bid2 = brewer_ids.reshape(N_ITEMS // 128, 128)
    wt = W.T
    cont_pad = jnp.pad(cont_feats, ((0, 0), (0, 8 - NF))).reshape(-1)

    mesh = plsc.VectorSubcoreMesh(core_axis_name="c", subcore_axis_name="s")
    f = pl.kernel(
        _encoder_body,
        out_type=jax.ShapeDtypeStruct((N_ITEMS, D), jnp.float32),
        mesh=mesh,
        compiler_params=pltpu.CompilerParams(
            needs_layout_passes=False, use_tc_tiling_on_sc=False),
        scratch_types=[
            pltpu.VMEM((NCH, CHUNK), jnp.int32),
            pltpu.VMEM((NCH, CHUNK), jnp.int32),
            pltpu.VMEM((BPW * 8 + 16,), jnp.float32),
            pltpu.VMEM((BPW, D), jnp.float32),
            pltpu.VMEM((BPW, D), jnp.float32),
            pltpu.VMEM((NF, D), jnp.float32),
            pltpu.VMEM((D,), jnp.float32),
            pltpu.VMEM((D,), jnp.float32),
            pltpu.VMEM((D,), jnp.float32),
            pltpu.VMEM((BPW, D), jnp.float32),
            pltpu.SemaphoreType.DMA,
            pltpu.SemaphoreType.DMA,
        ],
    )
    return f(sid2, bid2, cont_pad, style_table, brewer_table, wt, b, gamma, beta)
